# baseline (speedup 1.0000x reference)
import functools

import jax
import jax.numpy as jnp
from jax import lax
from jax.experimental import pallas as pl
from jax.experimental.pallas import tpu as pltpu
from jax.experimental.pallas import tpu_sc as plsc

_D = 64
_NW = 32
_C = 1040


def _sc_body(tvm_hbm, table_hbm, tw_hbm, vw_hbm, out_hbm,
             tvm_v, tab_v, tw_v, vw_v, o_v, nchunks, G):
    wid = lax.axis_index("s") * 2 + lax.axis_index("c")
    pltpu.sync_copy(table_hbm, tab_v)
    pltpu.sync_copy(tw_hbm, tw_v)
    pltpu.sync_copy(vw_hbm, vw_v)
    iota64 = lax.iota(jnp.int32, 16) * _D
    tw_q = [tw_v[pl.ds(q * 16, 16)] for q in range(4)]
    vw_q = [vw_v[pl.ds(q * 16, 16)] for q in range(4)]

    def chunk_body(j, carry):
        cid = wid * nchunks + j
        pltpu.sync_copy(tvm_hbm.at[cid], tvm_v)

        def group_body(g, c2):
            t16 = tvm_v[0, pl.ds(g * 16, 16)]
            v16 = tvm_v[1, pl.ds(g * 16, 16)]
            m16 = tvm_v[2, pl.ds(g * 16, 16)] > 0.5
            inv = jnp.isnan(v16)
            valid = jnp.logical_and(m16, jnp.logical_not(inv))
            v0 = jnp.where(valid, v16, 0.0)
            case = jnp.where(m16, jnp.where(inv, 2, 0), 1)
            idx_b = case * _D
            idx_o = iota64 + g * (16 * _D)
            for d in range(_D):
                bvec = plsc.load_gather(tab_v, [idx_b + d])
                col = t16 * tw_q[d // 16][d % 16] + v0 * vw_q[d // 16][d % 16] + bvec
                plsc.store_scatter(o_v, [idx_o + d], col)
            return c2

        lax.fori_loop(0, G, group_body, 0)
        pltpu.sync_copy(o_v, out_hbm.at[pl.ds(cid * (_C * _D), _C * _D)])
        return carry

    lax.fori_loop(0, nchunks, chunk_body, 0)


def kernel(x, monitor_mask, time_emb_w, time_emb_b, value_emb_w, value_emb_b,
           empty_token, unmonitored_token):
    N, T, P, _ = x.shape
    D = time_emb_w.shape[-1]
    R = N * T * P
    rpw = R // _NW
    nchunks = rpw // _C
    G = _C // 16

    value = x[..., 0].reshape(_NW * nchunks, 1, _C)
    time = x[..., 1].reshape(_NW * nchunks, 1, _C)
    maskf = monitor_mask.astype(jnp.float32).reshape(_NW * nchunks, 1, _C)
    tvm = jnp.concatenate([time, value, maskf], axis=1)

    tb = time_emb_b.reshape(D)
    table = jnp.concatenate([
        tb + value_emb_b.reshape(D),
        tb + unmonitored_token.reshape(D),
        tb + empty_token.reshape(D),
    ])

    mesh = plsc.VectorSubcoreMesh(core_axis_name="c", subcore_axis_name="s")
    body = functools.partial(_sc_body, nchunks=nchunks, G=G)
    out = pl.kernel(
        body,
        mesh=mesh,
        compiler_params=pltpu.CompilerParams(needs_layout_passes=False),
        out_type=jax.ShapeDtypeStruct((R * D,), jnp.float32),
        scratch_types=[
            pltpu.VMEM((3, _C), jnp.float32),
            pltpu.VMEM((3 * D,), jnp.float32),
            pltpu.VMEM((D,), jnp.float32),
            pltpu.VMEM((D,), jnp.float32),
            pltpu.VMEM((_C * D,), jnp.float32),
        ],
    )(tvm, table, time_emb_w.reshape(D), value_emb_w.reshape(D))
    return out.reshape(N, T, P, D)


# speedup vs baseline: 2.6608x; 2.6608x over previous
import functools

import jax
import jax.numpy as jnp
from jax import lax
from jax.experimental import pallas as pl
from jax.experimental.pallas import tpu as pltpu
from jax.experimental.pallas import tpu_sc as plsc

_D = 64
_NW = 32
_C = 720


def _prep_body(v_ref, t_ref, m_ref, vo_ref, to_ref, mo_ref, sem):
    pltpu.async_copy(v_ref, vo_ref, sem).wait()
    pltpu.async_copy(t_ref, to_ref, sem).wait()
    pltpu.async_copy(m_ref, mo_ref, sem).wait()


def _tc_prep(v2, t2, m2):
    M = v2.shape[0]
    spec = pl.BlockSpec(memory_space=pltpu.MemorySpace.HBM)
    return pl.pallas_call(
        _prep_body,
        in_specs=[spec, spec, spec],
        out_specs=[spec, spec, spec],
        out_shape=[jax.ShapeDtypeStruct((M, 128), jnp.float32)] * 3,
        scratch_shapes=[pltpu.SemaphoreType.DMA],
    )(v2, t2, m2)


def _sc_body(v_hbm, t_hbm, mf_hbm, table_hbm, tw_hbm, vw_hbm, out_hbm,
             vv_v, tv_v, mf_v, tab_v, tw_v, vw_v, o0_v, o1_v, sem0, sem1,
             nchunks, G):
    wid = lax.axis_index("s") * 2 + lax.axis_index("c")
    pltpu.sync_copy(table_hbm, tab_v)
    pltpu.sync_copy(tw_hbm, tw_v)
    pltpu.sync_copy(vw_hbm, vw_v)
    iota = lax.iota(jnp.int32, 16)
    iota_q = [iota + q * 16 for q in range(4)]
    tw_q = [tw_v[pl.ds(q * 16, 16)] for q in range(4)]
    vw_q = [vw_v[pl.ds(q * 16, 16)] for q in range(4)]

    def compute_chunk(j, o_v):
        cid = wid * nchunks + j
        start = cid * _C
        pltpu.sync_copy(v_hbm.at[pl.ds(start, _C)], vv_v)
        pltpu.sync_copy(t_hbm.at[pl.ds(start, _C)], tv_v)
        pltpu.sync_copy(mf_hbm.at[pl.ds(start, _C)], mf_v)

        @plsc.parallel_loop(0, G)
        def group_body(g):
            v16 = vv_v[pl.ds(g * 16, 16)]
            t16 = tv_v[pl.ds(g * 16, 16)]
            m16 = mf_v[pl.ds(g * 16, 16)] > 0.5
            inv = jnp.isnan(v16)
            valid = jnp.logical_and(m16, jnp.logical_not(inv))
            v0 = jnp.where(valid, v16, 0.0)
            case64 = jnp.where(m16, jnp.where(inv, 2, 0), 1) * _D
            gbase = g * (16 * _D)
            for r in range(16):
                c_b = case64[r]
                bvs = [plsc.load_gather(tab_v, [c_b + iq]) for iq in iota_q]
                t_b = t16[r]
                v_b = v0[r]
                tm = [t_b * w for w in tw_q]
                vm = [v_b * w for w in vw_q]
                sums = [a + b for a, b in zip(tm, vm)]
                cols = [s + b for s, b in zip(sums, bvs)]
                for q in range(4):
                    o_v[pl.ds(gbase + r * _D + q * 16, 16)] = cols[q]

        return cid

    def start_out(cid, o_v, sem):
        pltpu.async_copy(o_v, out_hbm.at[pl.ds(cid * (_C * _D), _C * _D)], sem)

    def drain(o_v, sem):
        pltpu.make_async_copy(o_v, out_hbm.at[pl.ds(0, _C * _D)], sem).wait()

    cid0 = compute_chunk(0, o0_v)
    start_out(cid0, o0_v, sem0)

    def pair_body(k, carry):
        cid1 = compute_chunk(2 * k + 1, o1_v)
        start_out(cid1, o1_v, sem1)
        drain(o0_v, sem0)
        cid2 = compute_chunk(2 * k + 2, o0_v)
        start_out(cid2, o0_v, sem0)
        drain(o1_v, sem1)
        return carry

    lax.fori_loop(0, (nchunks - 1) // 2, pair_body, 0)
    drain(o0_v, sem0)


def kernel(x, monitor_mask, time_emb_w, time_emb_b, value_emb_w, value_emb_b,
           empty_token, unmonitored_token):
    N, T, P, _ = x.shape
    D = time_emb_w.shape[-1]
    R = N * T * P
    rpw = R // _NW
    nchunks = rpw // _C
    G = _C // 16
    M = R // 128

    v2 = x[..., 0].reshape(M, 128)
    t2 = x[..., 1].reshape(M, 128)
    m2 = monitor_mask.astype(jnp.float32).reshape(M, 128)
    vp, tp, mp = _tc_prep(v2, t2, m2)

    tb = time_emb_b.reshape(D)
    table = jnp.concatenate([
        tb + value_emb_b.reshape(D),
        tb + unmonitored_token.reshape(D),
        tb + empty_token.reshape(D),
    ])

    mesh = plsc.VectorSubcoreMesh(core_axis_name="c", subcore_axis_name="s")
    body = functools.partial(_sc_body, nchunks=nchunks, G=G)
    out = pl.kernel(
        body,
        mesh=mesh,
        compiler_params=pltpu.CompilerParams(needs_layout_passes=False),
        out_type=jax.ShapeDtypeStruct((R * D,), jnp.float32),
        scratch_types=[
            pltpu.VMEM((_C,), jnp.float32),
            pltpu.VMEM((_C,), jnp.float32),
            pltpu.VMEM((_C,), jnp.float32),
            pltpu.VMEM((3 * D,), jnp.float32),
            pltpu.VMEM((D,), jnp.float32),
            pltpu.VMEM((D,), jnp.float32),
            pltpu.VMEM((_C * D,), jnp.float32),
            pltpu.VMEM((_C * D,), jnp.float32),
            pltpu.SemaphoreType.DMA,
            pltpu.SemaphoreType.DMA,
        ],
    )(vp.reshape(R), tp.reshape(R), mp.reshape(R), table,
      time_emb_w.reshape(D), value_emb_w.reshape(D))
    return out.reshape(N, T, P, D)


# speedup vs baseline: 9.3028x; 3.4963x over previous
import jax
import jax.numpy as jnp
from jax.experimental import pallas as pl
from jax.experimental.pallas import tpu as pltpu

_D = 64
_P = 325
_BT = 32


def _body(x_ref, m_ref, tw_ref, vw_ref, bval_ref, bunm_ref, bemp_ref, o_ref):
    v = x_ref[:, 0, :]
    t = x_ref[:, 1, :]
    m = m_ref[...]
    inv = jnp.isnan(v)
    valid = jnp.logical_and(m, jnp.logical_not(inv))
    v0 = jnp.where(valid, v, 0.0)
    for d in range(_D):
        base = jnp.where(m, jnp.where(inv, bemp_ref[d], bval_ref[d]), bunm_ref[d])
        o_ref[:, d, :] = t * tw_ref[d] + v0 * vw_ref[d] + base


def kernel(x, monitor_mask, time_emb_w, time_emb_b, value_emb_w, value_emb_b,
           empty_token, unmonitored_token):
    N, T, P, _ = x.shape
    D = time_emb_w.shape[-1]
    NT = N * T

    xt = x.transpose(0, 1, 3, 2).reshape(NT, 2, P)
    m2 = monitor_mask.reshape(NT, P)

    tb = time_emb_b.reshape(D)
    tw = time_emb_w.reshape(D)
    vw = value_emb_w.reshape(D)
    bval = tb + value_emb_b.reshape(D)
    bunm = tb + unmonitored_token.reshape(D)
    bemp = tb + empty_token.reshape(D)

    smem = pl.BlockSpec(memory_space=pltpu.MemorySpace.SMEM)
    out = pl.pallas_call(
        _body,
        grid=(NT // _BT,),
        in_specs=[
            pl.BlockSpec((_BT, 2, P), lambda i: (i, 0, 0)),
            pl.BlockSpec((_BT, P), lambda i: (i, 0)),
            smem, smem, smem, smem, smem,
        ],
        out_specs=pl.BlockSpec((_BT, D, P), lambda i: (i, 0, 0)),
        out_shape=jax.ShapeDtypeStruct((NT, D, P), jnp.float32),
    )(xt, m2, tw, vw, bval, bunm, bemp)
    return out.reshape(N, T, D, P).transpose(0, 1, 3, 2)


# speedup vs baseline: 11.8143x; 1.2700x over previous
import jax
import jax.numpy as jnp
from jax.experimental import pallas as pl
from jax.experimental.pallas import tpu as pltpu

_D = 64
_P = 325
_BT = 32


def _body(x_ref, m_ref, tw_ref, vw_ref, bval_ref, bunm_ref, bemp_ref, o_ref):
    v = x_ref[:, 0, :]
    t = x_ref[:, 1, :]
    m = m_ref[...]
    inv = jnp.isnan(v)
    valid = jnp.logical_and(m, jnp.logical_not(inv))
    v0 = jnp.where(valid, v, 0.0)
    mf = m.astype(jnp.float32)
    ef = jnp.where(valid, 0.0, mf)
    for d in range(_D):
        d1 = bval_ref[d] - bunm_ref[d]
        d2 = bemp_ref[d] - bval_ref[d]
        o_ref[:, d, :] = (t * tw_ref[d] + v0 * vw_ref[d]
                          + (mf * d1 + (ef * d2 + bunm_ref[d])))


def kernel(x, monitor_mask, time_emb_w, time_emb_b, value_emb_w, value_emb_b,
           empty_token, unmonitored_token):
    N, T, P, _ = x.shape
    D = time_emb_w.shape[-1]
    NT = N * T

    xt = x.transpose(0, 1, 3, 2).reshape(NT, 2, P)
    m2 = monitor_mask.reshape(NT, P)

    tb = time_emb_b.reshape(D)
    tw = time_emb_w.reshape(D)
    vw = value_emb_w.reshape(D)
    bval = tb + value_emb_b.reshape(D)
    bunm = tb + unmonitored_token.reshape(D)
    bemp = tb + empty_token.reshape(D)

    smem = pl.BlockSpec(memory_space=pltpu.MemorySpace.SMEM)
    out = pl.pallas_call(
        _body,
        grid=(NT // _BT,),
        in_specs=[
            pl.BlockSpec((_BT, 2, P), lambda i: (i, 0, 0)),
            pl.BlockSpec((_BT, P), lambda i: (i, 0)),
            smem, smem, smem, smem, smem,
        ],
        out_specs=pl.BlockSpec((_BT, D, P), lambda i: (i, 0, 0)),
        out_shape=jax.ShapeDtypeStruct((NT, D, P), jnp.float32),
    )(xt, m2, tw, vw, bval, bunm, bemp)
    return out.reshape(N, T, D, P).transpose(0, 1, 3, 2)
